# baseline (speedup 1.0000x reference)
import functools

import jax
import jax.numpy as jnp
from jax import lax
from jax.experimental import pallas as pl
from jax.experimental.pallas import tpu as pltpu
from jax.experimental.pallas import tpu_sc as plsc

_VOCAB1 = 1000001
_D = 8
_B = 16384
_H1 = 64
_H2 = 32

_CHUNK = 128
_LANES = 16

_RBLK = 8192
_RGRID = (_VOCAB1 + _RBLK - 1) // _RBLK
_FLAT = _RGRID * _RBLK * _D


def _reformat_body(tabt_ref, out_ref):
    x = tabt_ref[...]
    out_ref[...] = x.reshape(_D, _RBLK // 128, 128).transpose(1, 0, 2).reshape(
        _RBLK * _D
    )


def _reformat(tabt):
    return pl.pallas_call(
        _reformat_body,
        grid=(_RGRID,),
        in_specs=[pl.BlockSpec((_D, _RBLK), lambda i: (0, i))],
        out_specs=pl.BlockSpec((_RBLK * _D,), lambda i: (i,)),
        out_shape=jax.ShapeDtypeStruct((_FLAT,), jnp.float32),
    )(tabt)


def _gather_fn():
    info = plsc.get_sparse_core_info()
    nc, ns = info.num_cores, info.num_subcores
    nw = nc * ns
    b_per_w = _B // nw
    n_elem = b_per_w * _D
    n_chunks = n_elem // _CHUNK
    mesh = plsc.VectorSubcoreMesh(core_axis_name="c", subcore_axis_name="s")

    @functools.partial(
        pl.kernel,
        mesh=mesh,
        out_type=jax.ShapeDtypeStruct((_D * _B,), jnp.float32),
        scratch_types=[
            pltpu.VMEM((b_per_w,), jnp.int32),
            pltpu.VMEM((n_elem,), jnp.int32),
            pltpu.VMEM((n_elem,), jnp.float32),
            pltpu.SemaphoreType.DMA,
        ],
    )
    def gather(idx_hbm, tab_hbm, out_hbm, idx_v, pos_v, rows_v, sem):
        wid = lax.axis_index("s") * nc + lax.axis_index("c")
        base = wid * b_per_w
        pltpu.sync_copy(idx_hbm.at[pl.ds(base, b_per_w)], idx_v)
        for c in range(b_per_w // _LANES):
            v = idx_v[pl.ds(c * _LANES, _LANES)]
            p0 = lax.shift_left(lax.shift_right_logical(v, 7), 10) + lax.bitwise_and(
                v, 127
            )
            for j in range(_D):
                pos_v[pl.ds(j * b_per_w + c * _LANES, _LANES)] = p0 + j * 128
        copies = []
        for c in range(n_chunks):
            copies.append(
                pltpu.async_copy(
                    tab_hbm.at[pos_v.at[pl.ds(c * _CHUNK, _CHUNK)]],
                    rows_v.at[pl.ds(c * _CHUNK, _CHUNK)],
                    sem,
                )
            )
        for cp in copies:
            cp.wait()
        outs = []
        for j in range(_D):
            outs.append(
                pltpu.async_copy(
                    rows_v.at[pl.ds(j * b_per_w, b_per_w)],
                    out_hbm.at[pl.ds(j * _B + base, b_per_w)],
                    sem,
                )
            )
        for cp in outs:
            cp.wait()

    return gather


_gather = _gather_fn()


def _mlp_body(embt_ref, w1t_ref, b1_ref, w2t_ref, b2_ref, outt_ref):
    embt = embt_ref[...]
    h = jnp.dot(w1t_ref[...], embt, preferred_element_type=jnp.float32)
    h = jnp.maximum(h + b1_ref[...], 0.0)
    o = jnp.dot(w2t_ref[...], h, preferred_element_type=jnp.float32)
    outt_ref[...] = o + b2_ref[...]


def _mlp(embt, w1t, b1, w2t, b2):
    nblk = 8
    bn = _B // nblk
    return pl.pallas_call(
        _mlp_body,
        grid=(nblk,),
        in_specs=[
            pl.BlockSpec((_D, bn), lambda i: (0, i)),
            pl.BlockSpec((_H1, _D), lambda i: (0, 0)),
            pl.BlockSpec((_H1, 1), lambda i: (0, 0)),
            pl.BlockSpec((_H2, _H1), lambda i: (0, 0)),
            pl.BlockSpec((_H2, 1), lambda i: (0, 0)),
        ],
        out_specs=pl.BlockSpec((_H2, bn), lambda i: (0, i)),
        out_shape=jax.ShapeDtypeStruct((_H2, _B), jnp.float32),
    )(embt, w1t, b1, w2t, b2)


def kernel(indices, table, W1, b1, W2, b2):
    tab_flat = _reformat(table.T)
    embt = _gather(indices.astype(jnp.int32), tab_flat).reshape(_D, _B)
    outt = _mlp(
        embt,
        W1.T,
        b1.reshape(_H1, 1),
        W2.T,
        b2.reshape(_H2, 1),
    )
    return outt.T


# speedup vs baseline: 1.6343x; 1.6343x over previous
import functools

import jax
import jax.numpy as jnp
from jax import lax
from jax.experimental import pallas as pl
from jax.experimental.pallas import tpu as pltpu
from jax.experimental.pallas import tpu_sc as plsc

_VOCAB1 = 1000001
_D = 8
_B = 16384
_H1 = 64
_H2 = 32

_CHUNK = 128
_LANES = 16

_RBLK = 32768
_RGRID = (_VOCAB1 + _RBLK - 1) // _RBLK
_FLAT = _RGRID * _RBLK * _D


def _reformat_body(tabt_ref, out_ref):
    x = tabt_ref[...]
    out_ref[...] = x.reshape(_D, _RBLK // 128, 128).transpose(1, 0, 2).reshape(
        _RBLK * _D
    )


def _reformat(tabt):
    return pl.pallas_call(
        _reformat_body,
        grid=(_RGRID,),
        in_specs=[pl.BlockSpec((_D, _RBLK), lambda i: (0, i))],
        out_specs=pl.BlockSpec((_RBLK * _D,), lambda i: (i,)),
        out_shape=jax.ShapeDtypeStruct((_FLAT,), jnp.float32),
    )(tabt)


def _gather_fn():
    info = plsc.get_sparse_core_info()
    nc, ns = info.num_cores, info.num_subcores
    nw = nc * ns
    b_per_w = _B // nw
    n_elem = b_per_w * _D
    n_chunks = n_elem // _CHUNK
    mesh = plsc.VectorSubcoreMesh(core_axis_name="c", subcore_axis_name="s")

    @functools.partial(
        pl.kernel,
        mesh=mesh,
        out_type=jax.ShapeDtypeStruct((_D * _B,), jnp.float32),
        scratch_types=[
            pltpu.VMEM((b_per_w,), jnp.int32),
            pltpu.VMEM((n_elem,), jnp.int32),
            pltpu.VMEM((n_elem,), jnp.float32),
            pltpu.SemaphoreType.DMA,
        ],
    )
    def gather(idx_hbm, tab_hbm, out_hbm, idx_v, pos_v, rows_v, sem):
        wid = lax.axis_index("s") * nc + lax.axis_index("c")
        base = wid * b_per_w
        pltpu.sync_copy(idx_hbm.at[pl.ds(base, b_per_w)], idx_v)
        for c in range(b_per_w // _LANES):
            v = idx_v[pl.ds(c * _LANES, _LANES)]
            p0 = lax.shift_left(lax.shift_right_logical(v, 7), 10) + lax.bitwise_and(
                v, 127
            )
            for j in range(_D):
                pos_v[pl.ds(j * b_per_w + c * _LANES, _LANES)] = p0 + j * 128
        copies = []
        for c in range(n_chunks):
            copies.append(
                pltpu.async_copy(
                    tab_hbm.at[pos_v.at[pl.ds(c * _CHUNK, _CHUNK)]],
                    rows_v.at[pl.ds(c * _CHUNK, _CHUNK)],
                    sem,
                )
            )
        for cp in copies:
            cp.wait()
        outs = []
        for j in range(_D):
            outs.append(
                pltpu.async_copy(
                    rows_v.at[pl.ds(j * b_per_w, b_per_w)],
                    out_hbm.at[pl.ds(j * _B + base, b_per_w)],
                    sem,
                )
            )
        for cp in outs:
            cp.wait()

    return gather


_gather = _gather_fn()


def _mlp_body(embt_ref, w1t_ref, b1_ref, w2t_ref, b2_ref, outt_ref):
    embt = embt_ref[...]
    h = jnp.dot(w1t_ref[...], embt, preferred_element_type=jnp.float32)
    h = jnp.maximum(h + b1_ref[...], 0.0)
    o = jnp.dot(w2t_ref[...], h, preferred_element_type=jnp.float32)
    outt_ref[...] = o + b2_ref[...]


def _mlp(embt, w1t, b1, w2t, b2):
    nblk = 8
    bn = _B // nblk
    return pl.pallas_call(
        _mlp_body,
        grid=(nblk,),
        in_specs=[
            pl.BlockSpec((_D, bn), lambda i: (0, i)),
            pl.BlockSpec((_H1, _D), lambda i: (0, 0)),
            pl.BlockSpec((_H1, 1), lambda i: (0, 0)),
            pl.BlockSpec((_H2, _H1), lambda i: (0, 0)),
            pl.BlockSpec((_H2, 1), lambda i: (0, 0)),
        ],
        out_specs=pl.BlockSpec((_H2, bn), lambda i: (0, i)),
        out_shape=jax.ShapeDtypeStruct((_H2, _B), jnp.float32),
    )(embt, w1t, b1, w2t, b2)


def kernel(indices, table, W1, b1, W2, b2):
    tab_flat = _reformat(table.T)
    embt = _gather(indices.astype(jnp.int32), tab_flat).reshape(_D, _B)
    outt = _mlp(
        embt,
        W1.T,
        b1.reshape(_H1, 1),
        W2.T,
        b2.reshape(_H2, 1),
    )
    return outt.T


# speedup vs baseline: 1.8889x; 1.1558x over previous
import functools

import jax
import jax.numpy as jnp
from jax import lax
from jax.experimental import pallas as pl
from jax.experimental.pallas import tpu as pltpu
from jax.experimental.pallas import tpu_sc as plsc

_VOCAB1 = 1000001
_D = 8
_B = 16384
_H1 = 64
_H2 = 32

_CHUNK = 128
_LANES = 16

_RBLK = 65536
_RGRID = (_VOCAB1 + _RBLK - 1) // _RBLK
_FLAT = _RGRID * _RBLK * _D


def _reformat_body(tabt_ref, out_ref):
    x = tabt_ref[...]
    out_ref[...] = x.reshape(_D, _RBLK // 128, 128).transpose(1, 0, 2).reshape(
        _RBLK * _D
    )


def _reformat(tabt):
    return pl.pallas_call(
        _reformat_body,
        grid=(_RGRID,),
        in_specs=[pl.BlockSpec((_D, _RBLK), lambda i: (0, i))],
        out_specs=pl.BlockSpec((_RBLK * _D,), lambda i: (i,)),
        out_shape=jax.ShapeDtypeStruct((_FLAT,), jnp.float32),
    )(tabt)


def _gather_fn():
    info = plsc.get_sparse_core_info()
    nc, ns = info.num_cores, info.num_subcores
    nw = nc * ns
    b_per_w = _B // nw
    n_elem = b_per_w * _D
    n_chunks = n_elem // _CHUNK
    mesh = plsc.VectorSubcoreMesh(core_axis_name="c", subcore_axis_name="s")

    @functools.partial(
        pl.kernel,
        mesh=mesh,
        out_type=jax.ShapeDtypeStruct((_D * _B,), jnp.float32),
        scratch_types=[
            pltpu.VMEM((b_per_w,), jnp.int32),
            pltpu.VMEM((n_elem,), jnp.int32),
            pltpu.VMEM((n_elem,), jnp.float32),
            pltpu.SemaphoreType.DMA,
        ],
    )
    def gather(idx_hbm, tab_hbm, out_hbm, idx_v, pos_v, rows_v, sem):
        wid = lax.axis_index("s") * nc + lax.axis_index("c")
        base = wid * b_per_w
        pltpu.sync_copy(idx_hbm.at[pl.ds(base, b_per_w)], idx_v)
        for c in range(b_per_w // _LANES):
            v = idx_v[pl.ds(c * _LANES, _LANES)]
            p0 = lax.shift_left(lax.shift_right_logical(v, 7), 10) + lax.bitwise_and(
                v, 127
            )
            for j in range(_D):
                pos_v[pl.ds(j * b_per_w + c * _LANES, _LANES)] = p0 + j * 128
        copies = []
        for c in range(n_chunks):
            copies.append(
                pltpu.async_copy(
                    tab_hbm.at[pos_v.at[pl.ds(c * _CHUNK, _CHUNK)]],
                    rows_v.at[pl.ds(c * _CHUNK, _CHUNK)],
                    sem,
                )
            )
        for cp in copies:
            cp.wait()
        outs = []
        for j in range(_D):
            outs.append(
                pltpu.async_copy(
                    rows_v.at[pl.ds(j * b_per_w, b_per_w)],
                    out_hbm.at[pl.ds(j * _B + base, b_per_w)],
                    sem,
                )
            )
        for cp in outs:
            cp.wait()

    return gather


_gather = _gather_fn()


def _mlp_body(embt_ref, w1t_ref, b1_ref, w2t_ref, b2_ref, outt_ref):
    embt = embt_ref[...]
    h = jnp.dot(w1t_ref[...], embt, preferred_element_type=jnp.float32)
    h = jnp.maximum(h + b1_ref[...], 0.0)
    o = jnp.dot(w2t_ref[...], h, preferred_element_type=jnp.float32)
    outt_ref[...] = o + b2_ref[...]


def _mlp(embt, w1t, b1, w2t, b2):
    nblk = 8
    bn = _B // nblk
    return pl.pallas_call(
        _mlp_body,
        grid=(nblk,),
        in_specs=[
            pl.BlockSpec((_D, bn), lambda i: (0, i)),
            pl.BlockSpec((_H1, _D), lambda i: (0, 0)),
            pl.BlockSpec((_H1, 1), lambda i: (0, 0)),
            pl.BlockSpec((_H2, _H1), lambda i: (0, 0)),
            pl.BlockSpec((_H2, 1), lambda i: (0, 0)),
        ],
        out_specs=pl.BlockSpec((_H2, bn), lambda i: (0, i)),
        out_shape=jax.ShapeDtypeStruct((_H2, _B), jnp.float32),
    )(embt, w1t, b1, w2t, b2)


def kernel(indices, table, W1, b1, W2, b2):
    tab_flat = _reformat(table.T)
    embt = _gather(indices.astype(jnp.int32), tab_flat).reshape(_D, _B)
    outt = _mlp(
        embt,
        W1.T,
        b1.reshape(_H1, 1),
        W2.T,
        b2.reshape(_H2, 1),
    )
    return outt.T


# speedup vs baseline: 2.0009x; 1.0593x over previous
import functools

import jax
import jax.numpy as jnp
from jax import lax
from jax.experimental import pallas as pl
from jax.experimental.pallas import tpu as pltpu
from jax.experimental.pallas import tpu_sc as plsc

_VOCAB1 = 1000001
_D = 8
_B = 16384
_H1 = 64
_H2 = 32

_CHUNK = 128
_LANES = 16

_RBLK = 131072
_RGRID = (_VOCAB1 + _RBLK - 1) // _RBLK
_FLAT = _RGRID * _RBLK * _D


def _reformat_body(tabt_ref, out_ref):
    x = tabt_ref[...]
    out_ref[...] = x.reshape(_D, _RBLK // 128, 128).transpose(1, 0, 2).reshape(
        _RBLK * _D
    )


def _reformat(tabt):
    return pl.pallas_call(
        _reformat_body,
        grid=(_RGRID,),
        in_specs=[pl.BlockSpec((_D, _RBLK), lambda i: (0, i))],
        out_specs=pl.BlockSpec((_RBLK * _D,), lambda i: (i,)),
        out_shape=jax.ShapeDtypeStruct((_FLAT,), jnp.float32),
    )(tabt)


def _gather_fn():
    info = plsc.get_sparse_core_info()
    nc, ns = info.num_cores, info.num_subcores
    nw = nc * ns
    b_per_w = _B // nw
    n_elem = b_per_w * _D
    n_chunks = n_elem // _CHUNK
    mesh = plsc.VectorSubcoreMesh(core_axis_name="c", subcore_axis_name="s")

    @functools.partial(
        pl.kernel,
        mesh=mesh,
        out_type=jax.ShapeDtypeStruct((_D * _B,), jnp.float32),
        scratch_types=[
            pltpu.VMEM((b_per_w,), jnp.int32),
            pltpu.VMEM((n_elem,), jnp.int32),
            pltpu.VMEM((n_elem,), jnp.float32),
            pltpu.SemaphoreType.DMA,
        ],
    )
    def gather(idx_hbm, tab_hbm, out_hbm, idx_v, pos_v, rows_v, sem):
        wid = lax.axis_index("s") * nc + lax.axis_index("c")
        base = wid * b_per_w
        pltpu.sync_copy(idx_hbm.at[pl.ds(base, b_per_w)], idx_v)
        for c in range(b_per_w // _LANES):
            v = idx_v[pl.ds(c * _LANES, _LANES)]
            p0 = lax.shift_left(lax.shift_right_logical(v, 7), 10) + lax.bitwise_and(
                v, 127
            )
            for j in range(_D):
                pos_v[pl.ds(j * b_per_w + c * _LANES, _LANES)] = p0 + j * 128
        copies = []
        for c in range(n_chunks):
            copies.append(
                pltpu.async_copy(
                    tab_hbm.at[pos_v.at[pl.ds(c * _CHUNK, _CHUNK)]],
                    rows_v.at[pl.ds(c * _CHUNK, _CHUNK)],
                    sem,
                )
            )
        for cp in copies:
            cp.wait()
        outs = []
        for j in range(_D):
            outs.append(
                pltpu.async_copy(
                    rows_v.at[pl.ds(j * b_per_w, b_per_w)],
                    out_hbm.at[pl.ds(j * _B + base, b_per_w)],
                    sem,
                )
            )
        for cp in outs:
            cp.wait()

    return gather


_gather = _gather_fn()


def _mlp_body(embt_ref, w1t_ref, b1_ref, w2t_ref, b2_ref, outt_ref):
    embt = embt_ref[...]
    h = jnp.dot(w1t_ref[...], embt, preferred_element_type=jnp.float32)
    h = jnp.maximum(h + b1_ref[...], 0.0)
    o = jnp.dot(w2t_ref[...], h, preferred_element_type=jnp.float32)
    outt_ref[...] = o + b2_ref[...]


def _mlp(embt, w1t, b1, w2t, b2):
    nblk = 8
    bn = _B // nblk
    return pl.pallas_call(
        _mlp_body,
        grid=(nblk,),
        in_specs=[
            pl.BlockSpec((_D, bn), lambda i: (0, i)),
            pl.BlockSpec((_H1, _D), lambda i: (0, 0)),
            pl.BlockSpec((_H1, 1), lambda i: (0, 0)),
            pl.BlockSpec((_H2, _H1), lambda i: (0, 0)),
            pl.BlockSpec((_H2, 1), lambda i: (0, 0)),
        ],
        out_specs=pl.BlockSpec((_H2, bn), lambda i: (0, i)),
        out_shape=jax.ShapeDtypeStruct((_H2, _B), jnp.float32),
    )(embt, w1t, b1, w2t, b2)


def kernel(indices, table, W1, b1, W2, b2):
    tab_flat = _reformat(table.T)
    embt = _gather(indices.astype(jnp.int32), tab_flat).reshape(_D, _B)
    outt = _mlp(
        embt,
        W1.T,
        b1.reshape(_H1, 1),
        W2.T,
        b2.reshape(_H2, 1),
    )
    return outt.T
